# baseline (device time: 140013 ns/iter reference)
import jax
import jax.numpy as jnp
from jax import lax
from jax.experimental import pallas as pl
from jax.experimental.pallas import tpu as pltpu

N_DEV = 16
B = 2
SQ = 512
HQ_PER = 8
DH = 64
D_MODEL = 768
HD_PER = HQ_PER * DH
ROWS = B * SQ
CHUNK = ROWS // N_DEV
N_HOPS = 2 * (N_DEV - 1)


def _body(x_ref, wq_ref, k_ref, v_ref, wo_ref, out_ref,
          partial_ref, ctx_ref, send_ref, recv_ref,
          send_sems, recv_sems, credit_sem):
    my = lax.axis_index("i")
    left = lax.rem(my + N_DEV - 1, N_DEV)
    right = lax.rem(my + 1, N_DEV)

    barrier_sem = pltpu.get_barrier_semaphore()
    for nbr in (left, right):
        pl.semaphore_signal(barrier_sem, inc=1, device_id=(nbr,),
                            device_id_type=pl.DeviceIdType.MESH)
    pl.semaphore_wait(barrier_sem, 2)

    q = jnp.dot(x_ref[...], wq_ref[...],
                preferred_element_type=jnp.float32)

    row_blk = lax.broadcasted_iota(jnp.int32, (SQ, SQ), 0) // 64
    col_blk = lax.broadcasted_iota(jnp.int32, (SQ, SQ), 1) // 64
    mask = (row_blk % 4) == (col_blk % 4)

    for b in range(B):
        for h in range(HQ_PER):
            i = b * HQ_PER + h
            qbh = q[b * SQ:(b + 1) * SQ, h * DH:(h + 1) * DH]
            s = lax.dot_general(
                qbh.astype(jnp.bfloat16), k_ref[i],
                (((1,), (1,)), ((), ())),
                preferred_element_type=jnp.float32)
            s = jnp.where(mask, s * 0.125, -1e9)
            m = jnp.max(s, axis=1, keepdims=True)
            w = jnp.exp(s - m)
            w = w / jnp.sum(w, axis=1, keepdims=True)
            ctx = jnp.dot(w.astype(jnp.bfloat16), v_ref[i],
                          preferred_element_type=jnp.float32)
            ctx_ref[b * SQ:(b + 1) * SQ, h * DH:(h + 1) * DH] = (
                ctx.astype(jnp.bfloat16))

    partial_ref[...] = jnp.dot(ctx_ref[...], wo_ref[...],
                               preferred_element_type=jnp.float32)

    def rows_of(c):
        return pl.ds(c * CHUNK, CHUNK)

    def hop(hg, slot):
        if hg >= 2:
            pl.semaphore_wait(credit_sem, 1)
        rdma = pltpu.make_async_remote_copy(
            src_ref=send_ref.at[slot],
            dst_ref=recv_ref.at[slot],
            send_sem=send_sems.at[slot],
            recv_sem=recv_sems.at[slot],
            device_id=(right,),
            device_id_type=pl.DeviceIdType.MESH,
        )
        rdma.start()
        rdma.wait()

    def credit(hg):
        if hg + 2 <= N_HOPS - 1:
            pl.semaphore_signal(credit_sem, inc=1, device_id=(left,),
                                device_id_type=pl.DeviceIdType.MESH)

    send_ref[0, :, :] = partial_ref[rows_of(my), :]
    for h in range(N_DEV - 1):
        slot = h % 2
        hop(h, slot)
        rc = lax.rem(my + N_DEV - h - 1, N_DEV)
        summed = recv_ref[slot, :, :] + partial_ref[rows_of(rc), :]
        send_ref[1 - slot, :, :] = summed
        if h == N_DEV - 2:
            out_ref[rows_of(rc), :] = summed
        credit(h)

    for t in range(N_DEV - 1):
        hg = (N_DEV - 1) + t
        slot = hg % 2
        hop(hg, slot)
        gc = lax.rem(my + N_DEV - t, N_DEV)
        out_ref[rows_of(gc), :] = recv_ref[slot, :, :]
        if t < N_DEV - 2:
            send_ref[1 - slot, :, :] = recv_ref[slot, :, :]
        credit(hg)


def kernel(x, Wq, K_ext, V_ext, Wo):
    pos = lax.axis_index("i")
    x2 = x.reshape(ROWS, D_MODEL).astype(jnp.bfloat16)
    wq_i = lax.dynamic_slice(
        Wq, (0, pos * HD_PER), (D_MODEL, HD_PER)).astype(jnp.bfloat16)
    wo_i = lax.dynamic_slice(
        Wo, (pos * HD_PER, 0), (HD_PER, D_MODEL)).astype(jnp.bfloat16)
    k2 = K_ext.transpose(0, 2, 1, 3).reshape(
        B * HQ_PER, SQ, DH).astype(jnp.bfloat16)
    v2 = V_ext.transpose(0, 2, 1, 3).reshape(
        B * HQ_PER, SQ, DH).astype(jnp.bfloat16)

    out = pl.pallas_call(
        _body,
        out_shape=jax.ShapeDtypeStruct((ROWS, D_MODEL), jnp.float32),
        in_specs=[pl.BlockSpec(memory_space=pltpu.VMEM)] * 5,
        out_specs=pl.BlockSpec(memory_space=pltpu.VMEM),
        scratch_shapes=[
            pltpu.VMEM((ROWS, D_MODEL), jnp.float32),
            pltpu.VMEM((ROWS, HD_PER), jnp.bfloat16),
            pltpu.VMEM((2, CHUNK, D_MODEL), jnp.float32),
            pltpu.VMEM((2, CHUNK, D_MODEL), jnp.float32),
            pltpu.SemaphoreType.DMA((2,)),
            pltpu.SemaphoreType.DMA((2,)),
            pltpu.SemaphoreType.REGULAR,
        ],
        compiler_params=pltpu.CompilerParams(collective_id=0),
    )(x2, wq_i, k2, v2, wo_i)
    return out.reshape(B, SQ, D_MODEL)


# device time: 55238 ns/iter; 2.5347x vs baseline; 2.5347x over previous
import jax
import jax.numpy as jnp
from jax import lax
from jax.experimental import pallas as pl
from jax.experimental.pallas import tpu as pltpu

N_DEV = 16
B = 2
SQ = 512
HQ_PER = 8
DH = 64
D_MODEL = 768
HD_PER = HQ_PER * DH
ROWS = B * SQ
QROWS = ROWS // 4
ZROWS = QROWS // 4


def _body(x_ref, wq_ref, k_ref, v_ref, wo_ref, out_ref,
          ctx_ref, partial_ref, quar_ref, red_ref,
          p1_send, p1_recv, p2_send, p2_recv,
          p3a_send, p3a_recv, p3b_send, p3b_recv,
          p1_ss, p1_rs, p2_ss, p2_rs, p3a_ss, p3a_rs, p3b_ss, p3b_rs):
    my = lax.axis_index("i")
    r = lax.rem(my, 4)
    z = my // 4
    plane_base = z * 4

    def plane_mate(j):
        return plane_base + lax.rem(r + j, 4)

    def z_mate(j):
        return lax.rem(z + j, 4) * 4 + r

    barrier_sem = pltpu.get_barrier_semaphore()
    for j in (1, 2, 3):
        for tgt in (plane_mate(j), z_mate(j)):
            pl.semaphore_signal(barrier_sem, inc=1, device_id=(tgt,),
                                device_id_type=pl.DeviceIdType.MESH)
    pl.semaphore_wait(barrier_sem, 6)

    q = jnp.dot(x_ref[...], wq_ref[...],
                preferred_element_type=jnp.float32)

    row_blk = lax.broadcasted_iota(jnp.int32, (SQ, SQ), 0) // 64
    col_blk = lax.broadcasted_iota(jnp.int32, (SQ, SQ), 1) // 64
    mask = (row_blk % 4) == (col_blk % 4)

    for b in range(B):
        for h in range(HQ_PER):
            i = b * HQ_PER + h
            qbh = q[b * SQ:(b + 1) * SQ, h * DH:(h + 1) * DH]
            s = lax.dot_general(
                qbh.astype(jnp.bfloat16), k_ref[i],
                (((1,), (1,)), ((), ())),
                preferred_element_type=jnp.float32)
            s = jnp.where(mask, s * 0.125, -1e9)
            m = jnp.max(s, axis=1, keepdims=True)
            w = jnp.exp(s - m)
            w = w / jnp.sum(w, axis=1, keepdims=True)
            ctx = jnp.dot(w.astype(jnp.bfloat16), v_ref[i],
                          preferred_element_type=jnp.float32)
            ctx_ref[b * SQ:(b + 1) * SQ, h * DH:(h + 1) * DH] = (
                ctx.astype(jnp.bfloat16))

    partial_ref[...] = jnp.dot(ctx_ref[...], wo_ref[...],
                               preferred_element_type=jnp.float32)

    def quarter(qi):
        return partial_ref[pl.ds(qi * QROWS, QROWS), :]

    def rdma(src, dst, ssem, rsem, tgt):
        return pltpu.make_async_remote_copy(
            src_ref=src, dst_ref=dst, send_sem=ssem, recv_sem=rsem,
            device_id=(tgt,), device_id_type=pl.DeviceIdType.MESH)

    copies = []
    for j in (1, 2, 3):
        p1_send[j - 1, :, :] = quarter(lax.rem(r + j + 1, 4)
                                       ).astype(jnp.bfloat16)
        c = rdma(p1_send.at[j - 1], p1_recv.at[j - 1],
                 p1_ss.at[j - 1], p1_rs.at[j - 1], plane_mate(j))
        c.start()
        copies.append(c)
    qo = lax.rem(r + 1, 4)
    for c in copies:
        c.wait()
    quar_ref[...] = (quarter(qo)
                     + p1_recv[0].astype(jnp.float32)
                     + p1_recv[1].astype(jnp.float32)
                     + p1_recv[2].astype(jnp.float32))

    copies = []
    for j in (1, 2, 3):
        s_idx = lax.rem(z + j, 4)
        p2_send[j - 1, :, :] = quar_ref[pl.ds(s_idx * ZROWS, ZROWS), :
                                        ].astype(jnp.bfloat16)
        c = rdma(p2_send.at[j - 1], p2_recv.at[j - 1],
                 p2_ss.at[j - 1], p2_rs.at[j - 1], z_mate(j))
        c.start()
        copies.append(c)
    for c in copies:
        c.wait()
    red_ref[...] = (quar_ref[pl.ds(z * ZROWS, ZROWS), :]
                    + p2_recv[0].astype(jnp.float32)
                    + p2_recv[1].astype(jnp.float32)
                    + p2_recv[2].astype(jnp.float32))
    out_ref[pl.ds(qo * QROWS + z * ZROWS, ZROWS), :] = red_ref[...]
    p3a_send[...] = red_ref[...].astype(jnp.bfloat16)

    copies = []
    for j in (1, 2, 3):
        c = rdma(p3a_send, p3a_recv.at[j - 1],
                 p3a_ss.at[j - 1], p3a_rs.at[j - 1], z_mate(j))
        c.start()
        copies.append(c)
    p3b_send[pl.ds(z * ZROWS, ZROWS), :] = p3a_send[...]
    for c in copies:
        c.wait()
    for j in (1, 2, 3):
        s_idx = lax.rem(z + 4 - j, 4)
        out_ref[pl.ds(qo * QROWS + s_idx * ZROWS, ZROWS), :] = (
            p3a_recv[j - 1].astype(jnp.float32))
        p3b_send[pl.ds(s_idx * ZROWS, ZROWS), :] = p3a_recv[j - 1]

    copies = []
    for j in (1, 2, 3):
        c = rdma(p3b_send, p3b_recv.at[j - 1],
                 p3b_ss.at[j - 1], p3b_rs.at[j - 1], plane_mate(j))
        c.start()
        copies.append(c)
    for c in copies:
        c.wait()
    for j in (1, 2, 3):
        src_q = lax.rem(r + 4 - j + 1, 4)
        out_ref[pl.ds(src_q * QROWS, QROWS), :] = (
            p3b_recv[j - 1].astype(jnp.float32))


def kernel(x, Wq, K_ext, V_ext, Wo):
    pos = lax.axis_index("i")
    x2 = x.reshape(ROWS, D_MODEL).astype(jnp.bfloat16)
    wq_i = lax.dynamic_slice(
        Wq, (0, pos * HD_PER), (D_MODEL, HD_PER)).astype(jnp.bfloat16)
    wo_i = lax.dynamic_slice(
        Wo, (pos * HD_PER, 0), (HD_PER, D_MODEL)).astype(jnp.bfloat16)
    k2 = K_ext.transpose(0, 2, 1, 3).reshape(
        B * HQ_PER, SQ, DH).astype(jnp.bfloat16)
    v2 = V_ext.transpose(0, 2, 1, 3).reshape(
        B * HQ_PER, SQ, DH).astype(jnp.bfloat16)

    out = pl.pallas_call(
        _body,
        out_shape=jax.ShapeDtypeStruct((ROWS, D_MODEL), jnp.float32),
        in_specs=[pl.BlockSpec(memory_space=pltpu.VMEM)] * 5,
        out_specs=pl.BlockSpec(memory_space=pltpu.VMEM),
        scratch_shapes=[
            pltpu.VMEM((ROWS, HD_PER), jnp.bfloat16),
            pltpu.VMEM((ROWS, D_MODEL), jnp.float32),
            pltpu.VMEM((QROWS, D_MODEL), jnp.float32),
            pltpu.VMEM((ZROWS, D_MODEL), jnp.float32),
            pltpu.VMEM((3, QROWS, D_MODEL), jnp.bfloat16),
            pltpu.VMEM((3, QROWS, D_MODEL), jnp.bfloat16),
            pltpu.VMEM((3, ZROWS, D_MODEL), jnp.bfloat16),
            pltpu.VMEM((3, ZROWS, D_MODEL), jnp.bfloat16),
            pltpu.VMEM((ZROWS, D_MODEL), jnp.bfloat16),
            pltpu.VMEM((3, ZROWS, D_MODEL), jnp.bfloat16),
            pltpu.VMEM((QROWS, D_MODEL), jnp.bfloat16),
            pltpu.VMEM((3, QROWS, D_MODEL), jnp.bfloat16),
            pltpu.SemaphoreType.DMA((3,)),
            pltpu.SemaphoreType.DMA((3,)),
            pltpu.SemaphoreType.DMA((3,)),
            pltpu.SemaphoreType.DMA((3,)),
            pltpu.SemaphoreType.DMA((3,)),
            pltpu.SemaphoreType.DMA((3,)),
            pltpu.SemaphoreType.DMA((3,)),
            pltpu.SemaphoreType.DMA((3,)),
        ],
        compiler_params=pltpu.CompilerParams(collective_id=0),
    )(x2, wq_i, k2, v2, wo_i)
    return out.reshape(B, SQ, D_MODEL)


# device time: 46986 ns/iter; 2.9799x vs baseline; 1.1756x over previous
import jax
import jax.numpy as jnp
from jax import lax
from jax.experimental import pallas as pl
from jax.experimental.pallas import tpu as pltpu

COMPUTE_ONLY = False

N_DEV = 16
B = 2
SQ = 512
HQ_PER = 8
DH = 64
D_MODEL = 768
HD_PER = HQ_PER * DH
ROWS = B * SQ
HR = ROWS // 2
QR = HR // 4
ZR = QR // 4


def _body(x_ref, wq_ref, k_ref, v_ref, wo_ref, out_ref,
          ctx_ref, quar_ref, red_ref,
          p1_send, p1_recv, p2_send, p2_recv,
          p3a_send, p3a_recv, p3b_send, p3b_recv,
          p1_ss, p1_rs, p2_ss, p2_rs, p3a_ss, p3a_rs, p3b_ss, p3b_rs):
    my = lax.axis_index("i")
    r = lax.rem(my, 4)
    z = my // 4
    plane_base = z * 4

    def plane_mate(j):
        return plane_base + lax.rem(r + j, 4)

    def z_mate(j):
        return lax.rem(z + j, 4) * 4 + r

    if not COMPUTE_ONLY:
        barrier_sem = pltpu.get_barrier_semaphore()
        for j in (1, 2, 3):
            for tgt in (plane_mate(j), z_mate(j)):
                pl.semaphore_signal(barrier_sem, inc=1, device_id=(tgt,),
                                    device_id_type=pl.DeviceIdType.MESH)
        pl.semaphore_wait(barrier_sem, 6)

    q = jnp.dot(x_ref[...], wq_ref[...],
                preferred_element_type=jnp.float32)

    row_blk = lax.broadcasted_iota(jnp.int32, (SQ, SQ), 0) // 64
    col_blk = lax.broadcasted_iota(jnp.int32, (SQ, SQ), 1) // 64
    mask = (row_blk % 4) == (col_blk % 4)

    def partial_quarter(hf, qi):
        return jnp.dot(ctx_ref[pl.ds(hf * HR + qi * QR, QR), :], wo_ref[...],
                       preferred_element_type=jnp.float32)

    def rdma(src, dst, ssem, rsem, tgt):
        return pltpu.make_async_remote_copy(
            src_ref=src, dst_ref=dst, send_sem=ssem, recv_sem=rsem,
            device_id=(tgt,), device_id_type=pl.DeviceIdType.MESH)

    qo = lax.rem(r + 1, 4)
    HALVES = (0, 1)
    p1c = {0: [], 1: []}

    for b in range(B):
        for h in range(HQ_PER):
            i = b * HQ_PER + h
            qbh = q[b * SQ:(b + 1) * SQ, h * DH:(h + 1) * DH]
            s = lax.dot_general(
                qbh.astype(jnp.bfloat16), k_ref[i],
                (((1,), (1,)), ((), ())),
                preferred_element_type=jnp.float32)
            w = jnp.where(mask, jnp.exp(s), 0.0)
            denom = jnp.sum(w, axis=1, keepdims=True)
            ctx = jnp.dot(w.astype(jnp.bfloat16), v_ref[i],
                          preferred_element_type=jnp.float32) / denom
            ctx_ref[b * SQ:(b + 1) * SQ, h * DH:(h + 1) * DH] = (
                ctx.astype(jnp.bfloat16))
        if COMPUTE_ONLY:
            for qq in range(4):
                out_ref[b * HR + qq * QR:b * HR + (qq + 1) * QR, :] = (
                    partial_quarter(b, qq))
            continue
        for j in (1, 2, 3):
            p1_send[b, j - 1, :, :] = partial_quarter(
                b, lax.rem(r + j + 1, 4)).astype(jnp.bfloat16)
            c = rdma(p1_send.at[b, j - 1], p1_recv.at[b, j - 1],
                     p1_ss.at[b, j - 1], p1_rs.at[b, j - 1], plane_mate(j))
            c.start()
            p1c[b].append(c)

    if COMPUTE_ONLY:
        return

    own = {hf: partial_quarter(hf, qo) for hf in HALVES}

    p2c = {0: [], 1: []}
    for hf in HALVES:
        for c in p1c[hf]:
            c.wait()
        quar_ref[hf, :, :] = (own[hf]
                              + p1_recv[hf, 0].astype(jnp.float32)
                              + p1_recv[hf, 1].astype(jnp.float32)
                              + p1_recv[hf, 2].astype(jnp.float32))
        for j in (1, 2, 3):
            s_idx = lax.rem(z + j, 4)
            p2_send[hf, j - 1, :, :] = quar_ref[
                hf, pl.ds(s_idx * ZR, ZR), :].astype(jnp.bfloat16)
            c = rdma(p2_send.at[hf, j - 1], p2_recv.at[hf, j - 1],
                     p2_ss.at[hf, j - 1], p2_rs.at[hf, j - 1], z_mate(j))
            c.start()
            p2c[hf].append(c)

    p3ac = {0: [], 1: []}
    for hf in HALVES:
        for c in p2c[hf]:
            c.wait()
        red_ref[hf, :, :] = (quar_ref[hf, pl.ds(z * ZR, ZR), :]
                             + p2_recv[hf, 0].astype(jnp.float32)
                             + p2_recv[hf, 1].astype(jnp.float32)
                             + p2_recv[hf, 2].astype(jnp.float32))
        out_ref[pl.ds(hf * HR + qo * QR + z * ZR, ZR), :] = red_ref[hf]
        p3a_send[hf, :, :] = red_ref[hf].astype(jnp.bfloat16)
        for j in (1, 2, 3):
            c = rdma(p3a_send.at[hf], p3a_recv.at[hf, j - 1],
                     p3a_ss.at[hf, j - 1], p3a_rs.at[hf, j - 1], z_mate(j))
            c.start()
            p3ac[hf].append(c)

    p3bc = {0: [], 1: []}
    for hf in HALVES:
        for c in p3ac[hf]:
            c.wait()
        p3b_send[hf, pl.ds(z * ZR, ZR), :] = p3a_send[hf]
        for j in (1, 2, 3):
            s_idx = lax.rem(z + 4 - j, 4)
            out_ref[pl.ds(hf * HR + qo * QR + s_idx * ZR, ZR), :] = (
                p3a_recv[hf, j - 1].astype(jnp.float32))
            p3b_send[hf, pl.ds(s_idx * ZR, ZR), :] = p3a_recv[hf, j - 1]
        for j in (1, 2, 3):
            c = rdma(p3b_send.at[hf], p3b_recv.at[hf, j - 1],
                     p3b_ss.at[hf, j - 1], p3b_rs.at[hf, j - 1],
                     plane_mate(j))
            c.start()
            p3bc[hf].append(c)

    for hf in HALVES:
        for c in p3bc[hf]:
            c.wait()
        for j in (1, 2, 3):
            src_q = lax.rem(r + 4 - j + 1, 4)
            out_ref[pl.ds(hf * HR + src_q * QR, QR), :] = (
                p3b_recv[hf, j - 1].astype(jnp.float32))


def kernel(x, Wq, K_ext, V_ext, Wo):
    pos = lax.axis_index("i")
    x2 = x.reshape(ROWS, D_MODEL).astype(jnp.bfloat16)
    wq_i = (lax.dynamic_slice(Wq, (0, pos * HD_PER), (D_MODEL, HD_PER))
            * 0.125).astype(jnp.bfloat16)
    wo_i = lax.dynamic_slice(
        Wo, (pos * HD_PER, 0), (HD_PER, D_MODEL)).astype(jnp.bfloat16)
    k2 = K_ext.transpose(0, 2, 1, 3).reshape(
        B * HQ_PER, SQ, DH).astype(jnp.bfloat16)
    v2 = V_ext.transpose(0, 2, 1, 3).reshape(
        B * HQ_PER, SQ, DH).astype(jnp.bfloat16)

    out = pl.pallas_call(
        _body,
        out_shape=jax.ShapeDtypeStruct((ROWS, D_MODEL), jnp.float32),
        in_specs=[pl.BlockSpec(memory_space=pltpu.VMEM)] * 5,
        out_specs=pl.BlockSpec(memory_space=pltpu.VMEM),
        scratch_shapes=[
            pltpu.VMEM((ROWS, HD_PER), jnp.bfloat16),
            pltpu.VMEM((2, QR, D_MODEL), jnp.float32),
            pltpu.VMEM((2, ZR, D_MODEL), jnp.float32),
            pltpu.VMEM((2, 3, QR, D_MODEL), jnp.bfloat16),
            pltpu.VMEM((2, 3, QR, D_MODEL), jnp.bfloat16),
            pltpu.VMEM((2, 3, ZR, D_MODEL), jnp.bfloat16),
            pltpu.VMEM((2, 3, ZR, D_MODEL), jnp.bfloat16),
            pltpu.VMEM((2, ZR, D_MODEL), jnp.bfloat16),
            pltpu.VMEM((2, 3, ZR, D_MODEL), jnp.bfloat16),
            pltpu.VMEM((2, QR, D_MODEL), jnp.bfloat16),
            pltpu.VMEM((2, 3, QR, D_MODEL), jnp.bfloat16),
            pltpu.SemaphoreType.DMA((2, 3)),
            pltpu.SemaphoreType.DMA((2, 3)),
            pltpu.SemaphoreType.DMA((2, 3)),
            pltpu.SemaphoreType.DMA((2, 3)),
            pltpu.SemaphoreType.DMA((2, 3)),
            pltpu.SemaphoreType.DMA((2, 3)),
            pltpu.SemaphoreType.DMA((2, 3)),
            pltpu.SemaphoreType.DMA((2, 3)),
        ],
        compiler_params=(None if COMPUTE_ONLY
                         else pltpu.CompilerParams(collective_id=0)),
    )(x2, wq_i, k2, v2, wo_i)
    return out.reshape(B, SQ, D_MODEL)
